# baseline (speedup 1.0000x reference)
import jax
import jax.numpy as jnp
from jax import lax
from jax.experimental import pallas as pl
from jax.experimental.pallas import tpu as pltpu

E = 8
TOP_K = 2
D_IN = 1024
D_H = 4096
D_OUT = 1024

T_FFN = 256
T_ROUTE = 512
N_CHUNK = 8
CH = D_H // 4


def _routing_body(x_ref, mus_ref, sig_ref, lss_ref, lp_ref, w_ref, ti_ref, g_ref):
    x = x_ref[...]
    rows = x.shape[0]
    d = jnp.zeros((rows, E), jnp.float32)
    eidx = lax.broadcasted_iota(jnp.int32, (rows, E), 1)
    for e in range(E):
        t = (x - mus_ref[pl.ds(e, 1), :]) / sig_ref[pl.ds(e, 1), :]
        de = jnp.sum(t * t, axis=1, keepdims=True)
        d = jnp.where(eidx == e, de, d)
    lp = -0.5 * d - lss_ref[...]
    m = jnp.max(lp, axis=1, keepdims=True)
    ew = jnp.exp(lp - m)
    w = ew / jnp.sum(ew, axis=1, keepdims=True)

    iota = lax.broadcasted_iota(jnp.int32, w.shape, 1)
    m1 = jnp.max(w, axis=1, keepdims=True)
    i1 = jnp.min(jnp.where(w == m1, iota, E), axis=1, keepdims=True)
    wm = jnp.where(iota == i1, -jnp.inf, w)
    m2 = jnp.max(wm, axis=1, keepdims=True)
    i2 = jnp.min(jnp.where(wm == m2, iota, E), axis=1, keepdims=True)

    lp_ref[...] = lp
    w_ref[...] = w
    ti_ref[...] = jnp.where(iota == 0, i1, jnp.where(iota == 1, i2, 0))
    g_ref[...] = jnp.where(iota == 0, m1, jnp.where(iota == 1, m2, 0.0))


def _ffn_body(meta_ref, x_ref, b1_ref, b2_ref, w1_hbm, w2_hbm, y_ref,
              w1a, w1b, w2a, w2b, stage, sems, st):
    g = pl.program_id(0)
    te_g = meta_ref[0, g]
    first = meta_ref[1, g]
    slot = meta_ref[2, g]
    nxt_e = meta_ref[3, g]
    has_next = meta_ref[4, g]
    n_live = meta_ref[5, 0]

    def issue(c, e):
        b = lax.rem(c, 2)

        @pl.when(c < 4)
        def _():
            pltpu.make_async_copy(w1_hbm.at[e, :, c, :], stage.at[b],
                                  sems.at[b]).start()

        @pl.when(jnp.logical_and(c >= 4, c < N_CHUNK))
        def _():
            pltpu.make_async_copy(w2_hbm.at[e, c - 4, :, :], stage.at[b],
                                  sems.at[b]).start()

    def convert(c, to_slot_a, e):
        b = lax.rem(c, 2)
        pltpu.make_async_copy(w1_hbm.at[0, :, 0, :], stage.at[b],
                              sems.at[b]).wait()
        val = stage[b].astype(jnp.bfloat16)

        @pl.when(jnp.logical_and(c < 4, to_slot_a))
        def _():
            w1a[c] = val

        @pl.when(jnp.logical_and(c < 4, jnp.logical_not(to_slot_a)))
        def _():
            w1b[c] = val

        @pl.when(jnp.logical_and(c >= 4, to_slot_a))
        def _():
            w2a[c - 4] = val

        @pl.when(jnp.logical_and(c >= 4, jnp.logical_not(to_slot_a)))
        def _():
            w2b[c - 4] = val

        issue(c + 2, e)

    @pl.when(first == 1)
    def _():
        @pl.when(g == 0)
        def _():
            st[0] = 0
            issue(jnp.int32(0), te_g)
            issue(jnp.int32(1), te_g)

        def body(_, carry):
            cc = st[0]

            @pl.when(cc < N_CHUNK)
            def _():
                convert(cc, slot == 0, te_g)
                st[0] = cc + 1
            return carry

        lax.fori_loop(0, N_CHUNK, body, 0)
        st[0] = jnp.where(has_next == 1, 0, N_CHUNK)

        @pl.when(has_next == 1)
        def _():
            issue(jnp.int32(0), nxt_e)
            issue(jnp.int32(1), nxt_e)

    @pl.when(jnp.logical_and(first == 0, has_next == 1))
    def _():
        for _ in range(2):
            cc = st[0]

            @pl.when(cc < N_CHUNK)
            def _():
                convert(cc, slot != 0, nxt_e)
                st[0] = cc + 1

    def compute(w1s, w2s):
        x = x_ref[...]
        y = jnp.zeros((T_FFN, D_OUT), jnp.float32)
        for j in range(4):
            h = jnp.dot(x, w1s[j], preferred_element_type=jnp.float32)
            h = h + b1_ref[0, 0, pl.ds(j * CH, CH)]
            h = 0.5 * h * (1.0 + lax.erf(h * 0.7071067811865476))
            y = y + jnp.dot(h.astype(jnp.bfloat16), w2s[j],
                            preferred_element_type=jnp.float32)
        y_ref[...] = y + b2_ref[0]

    @pl.when(jnp.logical_and(g < n_live, slot == 0))
    def _():
        compute(w1a, w2a)

    @pl.when(jnp.logical_and(g < n_live, slot != 0))
    def _():
        compute(w1b, w2b)


def kernel(x, expert_mus, expert_log_sigmas, W1, b1, W2, b2):
    batch_size, num_tokens, _ = x.shape
    n = batch_size * num_tokens
    x_flat = x.reshape(n, D_IN)

    sigmas = jnp.exp(expert_log_sigmas)
    lss_row = jnp.sum(expert_log_sigmas, axis=-1).reshape(1, E)

    n_rt = n // T_ROUTE
    lp, w, ti_pad, g_pad = pl.pallas_call(
        _routing_body,
        grid=(n_rt,),
        in_specs=[
            pl.BlockSpec((T_ROUTE, D_IN), lambda i: (i, 0)),
            pl.BlockSpec((E, D_IN), lambda i: (0, 0)),
            pl.BlockSpec((E, D_IN), lambda i: (0, 0)),
            pl.BlockSpec((1, E), lambda i: (0, 0)),
        ],
        out_specs=[
            pl.BlockSpec((T_ROUTE, E), lambda i: (i, 0)),
            pl.BlockSpec((T_ROUTE, E), lambda i: (i, 0)),
            pl.BlockSpec((T_ROUTE, E), lambda i: (i, 0)),
            pl.BlockSpec((T_ROUTE, E), lambda i: (i, 0)),
        ],
        out_shape=[
            jax.ShapeDtypeStruct((n, E), jnp.float32),
            jax.ShapeDtypeStruct((n, E), jnp.float32),
            jax.ShapeDtypeStruct((n, E), jnp.int32),
            jax.ShapeDtypeStruct((n, E), jnp.float32),
        ],
    )(x_flat, expert_mus, sigmas, lss_row)

    top_indices = ti_pad[:, :TOP_K]
    gates = g_pad[:, :TOP_K]

    n_assign = n * TOP_K
    r_max = n_assign + E * T_FFN
    n_tiles = r_max // T_FFN

    e_flat = top_indices.reshape(-1)
    onehot = (e_flat[:, None] == jnp.arange(E, dtype=jnp.int32)[None, :])
    csum = jnp.cumsum(onehot.astype(jnp.int32), axis=0)
    counts = csum[-1]
    rank = jnp.take_along_axis(csum, e_flat[:, None], axis=1)[:, 0] - 1
    padded = ((counts + T_FFN - 1) // T_FFN) * T_FFN
    ends_pad = jnp.cumsum(padded)
    starts_pad = ends_pad - padded
    pos_a = starts_pad[e_flat] + rank

    token_a = jnp.arange(n_assign, dtype=jnp.int32) // TOP_K
    row_token = jnp.zeros((r_max,), jnp.int32).at[pos_a].set(token_a)
    te = jnp.clip(
        jnp.searchsorted(ends_pad, jnp.arange(n_tiles, dtype=jnp.int32) * T_FFN,
                         side="right"),
        0, E - 1).astype(jnp.int32)
    n_live = (ends_pad[-1] // T_FFN).astype(jnp.int32)

    first = jnp.concatenate([jnp.ones((1,), jnp.int32),
                             (te[1:] != te[:-1]).astype(jnp.int32)])
    run_idx = jnp.cumsum(first) - 1
    slot_par = run_idx % 2
    expert_by_run = jnp.zeros((n_tiles + 1,), jnp.int32).at[run_idx].set(te)
    nxt_e = expert_by_run[jnp.minimum(run_idx + 1, n_tiles)]
    has_next = (run_idx < run_idx[-1]).astype(jnp.int32)
    meta = jnp.stack([te, first, slot_par, nxt_e, has_next,
                      jnp.full((n_tiles,), 1, jnp.int32) * n_live])

    x_rows = x_flat.astype(jnp.bfloat16)[row_token]

    y = pl.pallas_call(
        _ffn_body,
        grid_spec=pltpu.PrefetchScalarGridSpec(
            num_scalar_prefetch=1,
            grid=(n_tiles,),
            in_specs=[
                pl.BlockSpec((T_FFN, D_IN), lambda g, m: (g, 0)),
                pl.BlockSpec((1, 1, D_H), lambda g, m: (m[0, g], 0, 0)),
                pl.BlockSpec((1, 1, D_OUT), lambda g, m: (m[0, g], 0, 0)),
                pl.BlockSpec(memory_space=pltpu.MemorySpace.HBM),
                pl.BlockSpec(memory_space=pltpu.MemorySpace.HBM),
            ],
            out_specs=pl.BlockSpec((T_FFN, D_OUT), lambda g, m: (g, 0)),
            scratch_shapes=[
                pltpu.VMEM((4, D_IN, CH), jnp.bfloat16),
                pltpu.VMEM((4, D_IN, CH), jnp.bfloat16),
                pltpu.VMEM((4, CH, D_OUT), jnp.bfloat16),
                pltpu.VMEM((4, CH, D_OUT), jnp.bfloat16),
                pltpu.VMEM((2, D_IN, CH), jnp.float32),
                pltpu.SemaphoreType.DMA((2,)),
                pltpu.SMEM((4,), jnp.int32),
            ],
        ),
        out_shape=jax.ShapeDtypeStruct((r_max, D_OUT), jnp.float32),
    )(meta, x_rows, b1.reshape(E, 1, D_H), b2.reshape(E, 1, D_OUT),
      W1.reshape(E, D_IN, 4, CH), W2.reshape(E, 4, CH, D_OUT))

    p0 = pos_a[0::TOP_K]
    p1 = pos_a[1::TOP_K]
    final = gates[:, 0:1] * y[p0] + gates[:, 1:2] * y[p1]

    return (final.reshape(batch_size, num_tokens, D_OUT),
            lp.reshape(batch_size, num_tokens, E),
            w.reshape(batch_size, num_tokens, E),
            top_indices)


# speedup vs baseline: 1.0283x; 1.0283x over previous
import jax
import jax.numpy as jnp
from jax import lax
from jax.experimental import pallas as pl
from jax.experimental.pallas import tpu as pltpu

E = 8
TOP_K = 2
D_IN = 1024
D_H = 4096
D_OUT = 1024

T_FFN = 256
T_ROUTE = 512
N_CHUNK = 8
CH = D_H // 4


def _routing_body(x_ref, mus_ref, sig_ref, lss_ref, lp_ref, w_ref, ti_ref, g_ref):
    x = x_ref[...]
    rows = x.shape[0]
    d = jnp.zeros((rows, E), jnp.float32)
    eidx = lax.broadcasted_iota(jnp.int32, (rows, E), 1)
    for e in range(E):
        t = (x - mus_ref[pl.ds(e, 1), :]) / sig_ref[pl.ds(e, 1), :]
        de = jnp.sum(t * t, axis=1, keepdims=True)
        d = jnp.where(eidx == e, de, d)
    lp = -0.5 * d - lss_ref[...]
    m = jnp.max(lp, axis=1, keepdims=True)
    ew = jnp.exp(lp - m)
    w = ew / jnp.sum(ew, axis=1, keepdims=True)

    iota = lax.broadcasted_iota(jnp.int32, w.shape, 1)
    m1 = jnp.max(w, axis=1, keepdims=True)
    i1 = jnp.min(jnp.where(w == m1, iota, E), axis=1, keepdims=True)
    wm = jnp.where(iota == i1, -jnp.inf, w)
    m2 = jnp.max(wm, axis=1, keepdims=True)
    i2 = jnp.min(jnp.where(wm == m2, iota, E), axis=1, keepdims=True)

    lp_ref[...] = lp
    w_ref[...] = w
    ti_ref[...] = jnp.where(iota == 0, i1, jnp.where(iota == 1, i2, 0))
    g_ref[...] = jnp.where(iota == 0, m1, jnp.where(iota == 1, m2, 0.0))


N_STAGE = 4


def _ffn_body(meta_ref, x_ref, b1_ref, b2_ref, w1_hbm, w2_hbm, y_ref,
              wbig, stage, sems, st):
    g = pl.program_id(0)
    te_g = meta_ref[0, g]
    first = meta_ref[1, g]
    slot = meta_ref[2, g]
    nxt_e = meta_ref[3, g]
    has_next = meta_ref[4, g]
    n_live = meta_ref[5, 0]

    def issue(c, e):
        b = lax.rem(c, N_STAGE)

        @pl.when(c < 4)
        def _():
            pltpu.make_async_copy(w1_hbm.at[e, :, c, :], stage.at[b],
                                  sems.at[b]).start()

        @pl.when(jnp.logical_and(c >= 4, c < N_CHUNK))
        def _():
            pltpu.make_async_copy(w2_hbm.at[e, c - 4, :, :], stage.at[b],
                                  sems.at[b]).start()

    def convert(c, fill_slot, e):
        b = lax.rem(c, N_STAGE)
        pltpu.make_async_copy(w1_hbm.at[0, :, 0, :], stage.at[b],
                              sems.at[b]).wait()
        wbig[fill_slot * N_CHUNK + c] = stage[b].astype(jnp.bfloat16)
        issue(c + N_STAGE, e)

    @pl.when(first == 1)
    def _():
        @pl.when(g == 0)
        def _():
            st[0] = 0
            for c0 in range(N_STAGE):
                issue(jnp.int32(c0), te_g)

        def body(_, carry):
            cc = st[0]

            @pl.when(cc < N_CHUNK)
            def _():
                convert(cc, slot, te_g)
                st[0] = cc + 1
            return carry

        lax.fori_loop(0, N_CHUNK, body, 0)
        st[0] = jnp.where(has_next == 1, 0, N_CHUNK)

        @pl.when(has_next == 1)
        def _():
            for c0 in range(N_STAGE):
                issue(jnp.int32(c0), nxt_e)

    @pl.when(g < n_live)
    def _():
        x = x_ref[...]
        y = jnp.zeros((T_FFN, D_OUT), jnp.float32)
        for j in range(4):
            h = jnp.dot(x, wbig[slot * N_CHUNK + j],
                        preferred_element_type=jnp.float32)
            h = h + b1_ref[0, 0, pl.ds(j * CH, CH)]
            h = 0.5 * h * (1.0 + lax.erf(h * 0.7071067811865476))
            y = y + jnp.dot(h.astype(jnp.bfloat16),
                            wbig[slot * N_CHUNK + 4 + j],
                            preferred_element_type=jnp.float32)
        y_ref[...] = y + b2_ref[0]

    @pl.when(jnp.logical_and(first == 0, has_next == 1))
    def _():
        for _ in range(2):
            cc = st[0]

            @pl.when(cc < N_CHUNK)
            def _():
                convert(cc, 1 - slot, nxt_e)
                st[0] = cc + 1


def kernel(x, expert_mus, expert_log_sigmas, W1, b1, W2, b2):
    batch_size, num_tokens, _ = x.shape
    n = batch_size * num_tokens
    x_flat = x.reshape(n, D_IN)

    sigmas = jnp.exp(expert_log_sigmas)
    lss_row = jnp.sum(expert_log_sigmas, axis=-1).reshape(1, E)

    n_rt = n // T_ROUTE
    lp, w, ti_pad, g_pad = pl.pallas_call(
        _routing_body,
        grid=(n_rt,),
        in_specs=[
            pl.BlockSpec((T_ROUTE, D_IN), lambda i: (i, 0)),
            pl.BlockSpec((E, D_IN), lambda i: (0, 0)),
            pl.BlockSpec((E, D_IN), lambda i: (0, 0)),
            pl.BlockSpec((1, E), lambda i: (0, 0)),
        ],
        out_specs=[
            pl.BlockSpec((T_ROUTE, E), lambda i: (i, 0)),
            pl.BlockSpec((T_ROUTE, E), lambda i: (i, 0)),
            pl.BlockSpec((T_ROUTE, E), lambda i: (i, 0)),
            pl.BlockSpec((T_ROUTE, E), lambda i: (i, 0)),
        ],
        out_shape=[
            jax.ShapeDtypeStruct((n, E), jnp.float32),
            jax.ShapeDtypeStruct((n, E), jnp.float32),
            jax.ShapeDtypeStruct((n, E), jnp.int32),
            jax.ShapeDtypeStruct((n, E), jnp.float32),
        ],
    )(x_flat, expert_mus, sigmas, lss_row)

    top_indices = ti_pad[:, :TOP_K]
    gates = g_pad[:, :TOP_K]

    n_assign = n * TOP_K
    r_max = n_assign + E * T_FFN
    n_tiles = r_max // T_FFN

    e_flat = top_indices.reshape(-1)
    onehot = (e_flat[:, None] == jnp.arange(E, dtype=jnp.int32)[None, :])
    csum = jnp.cumsum(onehot.astype(jnp.int32), axis=0)
    counts = csum[-1]
    rank = jnp.take_along_axis(csum, e_flat[:, None], axis=1)[:, 0] - 1
    padded = ((counts + T_FFN - 1) // T_FFN) * T_FFN
    ends_pad = jnp.cumsum(padded)
    starts_pad = ends_pad - padded
    pos_a = starts_pad[e_flat] + rank

    token_a = jnp.arange(n_assign, dtype=jnp.int32) // TOP_K
    row_token = jnp.zeros((r_max,), jnp.int32).at[pos_a].set(token_a)
    te = jnp.clip(
        jnp.searchsorted(ends_pad, jnp.arange(n_tiles, dtype=jnp.int32) * T_FFN,
                         side="right"),
        0, E - 1).astype(jnp.int32)
    n_live = (ends_pad[-1] // T_FFN).astype(jnp.int32)

    first = jnp.concatenate([jnp.ones((1,), jnp.int32),
                             (te[1:] != te[:-1]).astype(jnp.int32)])
    run_idx = jnp.cumsum(first) - 1
    slot_par = run_idx % 2
    expert_by_run = jnp.zeros((n_tiles + 1,), jnp.int32).at[run_idx].set(te)
    nxt_e = expert_by_run[jnp.minimum(run_idx + 1, n_tiles)]
    has_next = (run_idx < run_idx[-1]).astype(jnp.int32)
    meta = jnp.stack([te, first, slot_par, nxt_e, has_next,
                      jnp.full((n_tiles,), 1, jnp.int32) * n_live])

    x_rows = x_flat.astype(jnp.bfloat16)[row_token]

    y = pl.pallas_call(
        _ffn_body,
        grid_spec=pltpu.PrefetchScalarGridSpec(
            num_scalar_prefetch=1,
            grid=(n_tiles,),
            in_specs=[
                pl.BlockSpec((T_FFN, D_IN), lambda g, m: (g, 0)),
                pl.BlockSpec((1, 1, D_H), lambda g, m: (m[0, g], 0, 0)),
                pl.BlockSpec((1, 1, D_OUT), lambda g, m: (m[0, g], 0, 0)),
                pl.BlockSpec(memory_space=pltpu.MemorySpace.HBM),
                pl.BlockSpec(memory_space=pltpu.MemorySpace.HBM),
            ],
            out_specs=pl.BlockSpec((T_FFN, D_OUT), lambda g, m: (g, 0)),
            scratch_shapes=[
                pltpu.VMEM((2 * N_CHUNK, D_IN, CH), jnp.bfloat16),
                pltpu.VMEM((N_STAGE, D_IN, CH), jnp.float32),
                pltpu.SemaphoreType.DMA((N_STAGE,)),
                pltpu.SMEM((4,), jnp.int32),
            ],
        ),
        out_shape=jax.ShapeDtypeStruct((r_max, D_OUT), jnp.float32),
    )(meta, x_rows, b1.reshape(E, 1, D_H), b2.reshape(E, 1, D_OUT),
      W1.reshape(E, D_IN, 4, CH), W2.reshape(E, 4, CH, D_OUT))

    p0 = pos_a[0::TOP_K]
    p1 = pos_a[1::TOP_K]
    final = gates[:, 0:1] * y[p0] + gates[:, 1:2] * y[p1]

    return (final.reshape(batch_size, num_tokens, D_OUT),
            lp.reshape(batch_size, num_tokens, E),
            w.reshape(batch_size, num_tokens, E),
            top_indices)


# speedup vs baseline: 1.1502x; 1.1186x over previous
import functools

import jax
import jax.numpy as jnp
from jax.experimental import pallas as pl
from jax.experimental.pallas import tpu as pltpu

E = 8
TOP_K = 2
D_IN = 1024
D_H = 4096
D_OUT = 1024

T_FFN = 512
T_ROUTE = 512


def _routing_body(x_ref, mus_ref, sig_ref, lss_ref, lp_ref, w_ref, ti_ref, g_ref):
    x = x_ref[...]
    rows = x.shape[0]
    d = jnp.zeros((rows, E), jnp.float32)
    eidx = jax.lax.broadcasted_iota(jnp.int32, (rows, E), 1)
    for e in range(E):
        t = (x - mus_ref[pl.ds(e, 1), :]) / sig_ref[pl.ds(e, 1), :]
        de = jnp.sum(t * t, axis=1, keepdims=True)
        d = jnp.where(eidx == e, de, d)
    lp = -0.5 * d - lss_ref[...]
    m = jnp.max(lp, axis=1, keepdims=True)
    ew = jnp.exp(lp - m)
    w = ew / jnp.sum(ew, axis=1, keepdims=True)

    iota = jax.lax.broadcasted_iota(jnp.int32, w.shape, 1)
    m1 = jnp.max(w, axis=1, keepdims=True)
    i1 = jnp.min(jnp.where(w == m1, iota, E), axis=1, keepdims=True)
    wm = jnp.where(iota == i1, -jnp.inf, w)
    m2 = jnp.max(wm, axis=1, keepdims=True)
    i2 = jnp.min(jnp.where(wm == m2, iota, E), axis=1, keepdims=True)

    lp_ref[...] = lp
    w_ref[...] = w
    ti_ref[...] = jnp.where(iota == 0, i1, jnp.where(iota == 1, i2, 0))
    g_ref[...] = jnp.where(iota == 0, m1, jnp.where(iota == 1, m2, 0.0))


def _ffn_body(te_ref, x_ref, w1_ref, b1_ref, w2_ref, b2_ref, y_ref):
    n_tiles = pl.num_programs(0)

    @pl.when(pl.program_id(0) < te_ref[n_tiles])
    def _():
        h = jnp.dot(x_ref[...], w1_ref[0], preferred_element_type=jnp.float32)
        h = h + b1_ref[0]
        h = 0.5 * h * (1.0 + jax.lax.erf(h * 0.7071067811865476))
        y = jnp.dot(h.astype(jnp.bfloat16), w2_ref[0],
                    preferred_element_type=jnp.float32)
        y_ref[...] = y + b2_ref[0]


def kernel(x, expert_mus, expert_log_sigmas, W1, b1, W2, b2):
    batch_size, num_tokens, _ = x.shape
    n = batch_size * num_tokens
    x_flat = x.reshape(n, D_IN)

    sigmas = jnp.exp(expert_log_sigmas)
    lss_row = jnp.sum(expert_log_sigmas, axis=-1).reshape(1, E)

    n_rt = n // T_ROUTE
    lp, w, ti_pad, g_pad = pl.pallas_call(
        _routing_body,
        grid=(n_rt,),
        in_specs=[
            pl.BlockSpec((T_ROUTE, D_IN), lambda i: (i, 0)),
            pl.BlockSpec((E, D_IN), lambda i: (0, 0)),
            pl.BlockSpec((E, D_IN), lambda i: (0, 0)),
            pl.BlockSpec((1, E), lambda i: (0, 0)),
        ],
        out_specs=[
            pl.BlockSpec((T_ROUTE, E), lambda i: (i, 0)),
            pl.BlockSpec((T_ROUTE, E), lambda i: (i, 0)),
            pl.BlockSpec((T_ROUTE, E), lambda i: (i, 0)),
            pl.BlockSpec((T_ROUTE, E), lambda i: (i, 0)),
        ],
        out_shape=[
            jax.ShapeDtypeStruct((n, E), jnp.float32),
            jax.ShapeDtypeStruct((n, E), jnp.float32),
            jax.ShapeDtypeStruct((n, E), jnp.int32),
            jax.ShapeDtypeStruct((n, E), jnp.float32),
        ],
    )(x_flat, expert_mus, sigmas, lss_row)

    top_indices = ti_pad[:, :TOP_K]
    gates = g_pad[:, :TOP_K]

    n_assign = n * TOP_K
    r_max = n_assign + E * T_FFN
    n_tiles = r_max // T_FFN

    e_flat = top_indices.reshape(-1)
    onehot = (e_flat[:, None] == jnp.arange(E, dtype=jnp.int32)[None, :])
    csum = jnp.cumsum(onehot.astype(jnp.int32), axis=0)
    counts = csum[-1]
    rank = jnp.take_along_axis(csum, e_flat[:, None], axis=1)[:, 0] - 1
    padded = ((counts + T_FFN - 1) // T_FFN) * T_FFN
    ends_pad = jnp.cumsum(padded)
    starts_pad = ends_pad - padded
    pos_a = starts_pad[e_flat] + rank

    token_a = jnp.arange(n_assign, dtype=jnp.int32) // TOP_K
    row_token = jnp.zeros((r_max,), jnp.int32).at[pos_a].set(token_a)
    tile_expert = jnp.clip(
        jnp.searchsorted(ends_pad, jnp.arange(n_tiles, dtype=jnp.int32) * T_FFN,
                         side="right"),
        0, E - 1).astype(jnp.int32)
    n_live = (ends_pad[-1] // T_FFN).astype(jnp.int32)
    tile_meta = jnp.concatenate([tile_expert, n_live[None]])

    x_rows = x_flat.astype(jnp.bfloat16)[row_token]

    y = pl.pallas_call(
        _ffn_body,
        grid_spec=pltpu.PrefetchScalarGridSpec(
            num_scalar_prefetch=1,
            grid=(n_tiles,),
            in_specs=[
                pl.BlockSpec((T_FFN, D_IN), lambda g, te: (g, 0)),
                pl.BlockSpec((1, D_IN, D_H), lambda g, te: (te[g], 0, 0)),
                pl.BlockSpec((1, 1, D_H), lambda g, te: (te[g], 0, 0)),
                pl.BlockSpec((1, D_H, D_OUT), lambda g, te: (te[g], 0, 0)),
                pl.BlockSpec((1, 1, D_OUT), lambda g, te: (te[g], 0, 0)),
            ],
            out_specs=pl.BlockSpec((T_FFN, D_OUT), lambda g, te: (g, 0)),
        ),
        out_shape=jax.ShapeDtypeStruct((r_max, D_OUT), jnp.float32),
    )(tile_meta, x_rows, W1.astype(jnp.bfloat16), b1.reshape(E, 1, D_H),
      W2.astype(jnp.bfloat16), b2.reshape(E, 1, D_OUT))

    p0 = pos_a[0::TOP_K]
    p1 = pos_a[1::TOP_K]
    final = gates[:, 0:1] * y[p0] + gates[:, 1:2] * y[p1]

    return (final.reshape(batch_size, num_tokens, D_OUT),
            lp.reshape(batch_size, num_tokens, E),
            w.reshape(batch_size, num_tokens, E),
            top_indices)


# speedup vs baseline: 1.2073x; 1.0496x over previous
import jax
import jax.numpy as jnp
from jax import lax
from jax.experimental import pallas as pl
from jax.experimental.pallas import tpu as pltpu

E = 8
TOP_K = 2
D_IN = 1024
D_H = 4096
D_OUT = 1024

T_FFN = 512
T_ROUTE = 512


def _routing_body(x_ref, mus_ref, sig_ref, lss_ref, lp_ref, w_ref, ti_ref, g_ref,
                  xb_ref):
    x = x_ref[...]
    rows = x.shape[0]
    d = jnp.zeros((rows, E), jnp.float32)
    eidx = jax.lax.broadcasted_iota(jnp.int32, (rows, E), 1)
    for e in range(E):
        t = (x - mus_ref[pl.ds(e, 1), :]) / sig_ref[pl.ds(e, 1), :]
        de = jnp.sum(t * t, axis=1, keepdims=True)
        d = jnp.where(eidx == e, de, d)
    lp = -0.5 * d - lss_ref[...]
    m = jnp.max(lp, axis=1, keepdims=True)
    ew = jnp.exp(lp - m)
    w = ew / jnp.sum(ew, axis=1, keepdims=True)

    iota = jax.lax.broadcasted_iota(jnp.int32, w.shape, 1)
    m1 = jnp.max(w, axis=1, keepdims=True)
    i1 = jnp.min(jnp.where(w == m1, iota, E), axis=1, keepdims=True)
    wm = jnp.where(iota == i1, -jnp.inf, w)
    m2 = jnp.max(wm, axis=1, keepdims=True)
    i2 = jnp.min(jnp.where(wm == m2, iota, E), axis=1, keepdims=True)

    lp_ref[...] = lp
    w_ref[...] = w
    ti_ref[...] = jnp.where(iota == 0, i1, jnp.where(iota == 1, i2, 0))
    g_ref[...] = jnp.where(iota == 0, m1, jnp.where(iota == 1, m2, 0.0))
    xb_ref[...] = x.astype(jnp.bfloat16)


def _dispatch_body(ti_ref, pos_ref, meta_ref):
    C = 512
    n = ti_ref.shape[0]
    e0 = ti_ref[:, 0:1]
    e1 = ti_ref[:, 1:2]
    lane8 = lax.broadcasted_iota(jnp.int32, (n, E), 1)
    oh0 = (e0 == lane8).astype(jnp.float32)
    oh1 = (e1 == lane8).astype(jnp.float32)
    oh = oh0 + oh1

    r = lax.broadcasted_iota(jnp.int32, (C, C), 0)
    c = lax.broadcasted_iota(jnp.int32, (C, C), 1)
    L = (r > c).astype(jnp.float32)

    carry = jnp.zeros((1, E), jnp.float32)
    chunks = []
    for k in range(n // C):
        ohk = oh[k * C:(k + 1) * C, :]
        pref = jnp.dot(L, ohk, preferred_element_type=jnp.float32) + carry
        chunks.append(pref)
        carry = carry + jnp.sum(ohk, axis=0, keepdims=True)
    prefix = jnp.concatenate(chunks, axis=0)
    counts = carry

    padded = jnp.ceil(counts / T_FFN) * T_FFN
    u_r = lax.broadcasted_iota(jnp.int32, (E, E), 0)
    u_c = lax.broadcasted_iota(jnp.int32, (E, E), 1)
    U = (u_r < u_c).astype(jnp.float32)
    starts = jnp.dot(padded, U, preferred_element_type=jnp.float32)
    ends = starts + padded

    rank0 = jnp.sum(prefix * oh0, axis=1, keepdims=True)
    rank1 = jnp.sum((prefix + oh0) * oh1, axis=1, keepdims=True)
    base0 = jnp.sum(starts * oh0, axis=1, keepdims=True)
    base1 = jnp.sum(starts * oh1, axis=1, keepdims=True)
    pos0 = (base0 + rank0).astype(jnp.int32)
    pos1 = (base1 + rank1).astype(jnp.int32)
    lane_out = lax.broadcasted_iota(jnp.int32, (n, E), 1)
    pos_ref[...] = jnp.where(lane_out == 0, pos0,
                             jnp.where(lane_out == 1, pos1, 0))

    n_tiles = (pos_ref.shape[0] * TOP_K + E * T_FFN) // T_FFN
    gl = lax.broadcasted_iota(jnp.int32, (1, 128), 1)
    ends_b = jnp.broadcast_to(ends.reshape(E, 1), (E, 128))
    te_row = jnp.minimum(
        jnp.sum((ends_b <= (gl * T_FFN).astype(jnp.float32)).astype(jnp.int32),
                axis=0, keepdims=True), E - 1)
    n_live = (ends[0, E - 1] / T_FFN).astype(jnp.int32)
    meta_ref[...] = jnp.where(gl == n_tiles, n_live, te_row)


def _ffn_body(te_ref, x_ref, w1_ref, b1_ref, w2_ref, b2_ref, y_ref):
    n_tiles = pl.num_programs(0)

    @pl.when(pl.program_id(0) < te_ref[n_tiles])
    def _():
        h = jnp.dot(x_ref[...], w1_ref[0], preferred_element_type=jnp.float32)
        h = h + b1_ref[0]
        h = 0.5 * h * (1.0 + jax.lax.erf(h * 0.7071067811865476))
        y = jnp.dot(h.astype(jnp.bfloat16), w2_ref[0],
                    preferred_element_type=jnp.float32)
        y_ref[...] = y + b2_ref[0]


def kernel(x, expert_mus, expert_log_sigmas, W1, b1, W2, b2):
    batch_size, num_tokens, _ = x.shape
    n = batch_size * num_tokens
    x_flat = x.reshape(n, D_IN)

    sigmas = jnp.exp(expert_log_sigmas)
    lss_row = jnp.sum(expert_log_sigmas, axis=-1).reshape(1, E)

    n_rt = n // T_ROUTE
    lp, w, ti_pad, g_pad, xb = pl.pallas_call(
        _routing_body,
        grid=(n_rt,),
        in_specs=[
            pl.BlockSpec((T_ROUTE, D_IN), lambda i: (i, 0)),
            pl.BlockSpec((E, D_IN), lambda i: (0, 0)),
            pl.BlockSpec((E, D_IN), lambda i: (0, 0)),
            pl.BlockSpec((1, E), lambda i: (0, 0)),
        ],
        out_specs=[
            pl.BlockSpec((T_ROUTE, E), lambda i: (i, 0)),
            pl.BlockSpec((T_ROUTE, E), lambda i: (i, 0)),
            pl.BlockSpec((T_ROUTE, E), lambda i: (i, 0)),
            pl.BlockSpec((T_ROUTE, E), lambda i: (i, 0)),
            pl.BlockSpec((T_ROUTE, D_IN), lambda i: (i, 0)),
        ],
        out_shape=[
            jax.ShapeDtypeStruct((n, E), jnp.float32),
            jax.ShapeDtypeStruct((n, E), jnp.float32),
            jax.ShapeDtypeStruct((n, E), jnp.int32),
            jax.ShapeDtypeStruct((n, E), jnp.float32),
            jax.ShapeDtypeStruct((n, D_IN), jnp.bfloat16),
        ],
    )(x_flat, expert_mus, sigmas, lss_row)

    top_indices = ti_pad[:, :TOP_K]
    gates = g_pad[:, :TOP_K]

    n_assign = n * TOP_K
    r_max = n_assign + E * T_FFN
    n_tiles = r_max // T_FFN

    pos_pad, meta_row = pl.pallas_call(
        _dispatch_body,
        out_shape=[
            jax.ShapeDtypeStruct((n, E), jnp.int32),
            jax.ShapeDtypeStruct((1, 128), jnp.int32),
        ],
    )(ti_pad)

    pos_a = pos_pad[:, :TOP_K].reshape(-1)
    token_a = jnp.arange(n_assign, dtype=jnp.int32) // TOP_K
    row_token = jnp.zeros((r_max,), jnp.int32).at[pos_a].set(token_a)
    tile_meta = meta_row[0, :n_tiles + 1]

    x_rows = xb[row_token]

    y = pl.pallas_call(
        _ffn_body,
        grid_spec=pltpu.PrefetchScalarGridSpec(
            num_scalar_prefetch=1,
            grid=(n_tiles,),
            in_specs=[
                pl.BlockSpec((T_FFN, D_IN), lambda g, te: (g, 0)),
                pl.BlockSpec((1, D_IN, D_H), lambda g, te: (te[g], 0, 0)),
                pl.BlockSpec((1, 1, D_H), lambda g, te: (te[g], 0, 0)),
                pl.BlockSpec((1, D_H, D_OUT), lambda g, te: (te[g], 0, 0)),
                pl.BlockSpec((1, 1, D_OUT), lambda g, te: (te[g], 0, 0)),
            ],
            out_specs=pl.BlockSpec((T_FFN, D_OUT), lambda g, te: (g, 0)),
        ),
        out_shape=jax.ShapeDtypeStruct((r_max, D_OUT), jnp.float32),
    )(tile_meta, x_rows, W1.astype(jnp.bfloat16), b1.reshape(E, 1, D_H),
      W2.astype(jnp.bfloat16), b2.reshape(E, 1, D_OUT))

    p0 = pos_a[0::TOP_K]
    p1 = pos_a[1::TOP_K]
    final = gates[:, 0:1] * y[p0] + gates[:, 1:2] * y[p1]

    return (final.reshape(batch_size, num_tokens, D_OUT),
            lp.reshape(batch_size, num_tokens, E),
            w.reshape(batch_size, num_tokens, E),
            top_indices)


# speedup vs baseline: 1.2089x; 1.0014x over previous
import jax
import jax.numpy as jnp
from jax import lax
from jax.experimental import pallas as pl
from jax.experimental.pallas import tpu as pltpu

E = 8
TOP_K = 2
D_IN = 1024
D_H = 4096
D_OUT = 1024

T_FFN = 512
T_ROUTE = 512


def _routing_body(x_ref, mus_ref, sig_ref, lss_ref, lp_ref, w_ref, ti_ref, g_ref,
                  xb_ref):
    x = x_ref[...]
    rows = x.shape[0]
    d = jnp.zeros((rows, E), jnp.float32)
    eidx = jax.lax.broadcasted_iota(jnp.int32, (rows, E), 1)
    for e in range(E):
        t = (x - mus_ref[pl.ds(e, 1), :]) * sig_ref[pl.ds(e, 1), :]
        de = jnp.sum(t * t, axis=1, keepdims=True)
        d = jnp.where(eidx == e, de, d)
    lp = -0.5 * d - lss_ref[...]
    m = jnp.max(lp, axis=1, keepdims=True)
    ew = jnp.exp(lp - m)
    w = ew / jnp.sum(ew, axis=1, keepdims=True)

    iota = jax.lax.broadcasted_iota(jnp.int32, w.shape, 1)
    m1 = jnp.max(w, axis=1, keepdims=True)
    i1 = jnp.min(jnp.where(w == m1, iota, E), axis=1, keepdims=True)
    wm = jnp.where(iota == i1, -jnp.inf, w)
    m2 = jnp.max(wm, axis=1, keepdims=True)
    i2 = jnp.min(jnp.where(wm == m2, iota, E), axis=1, keepdims=True)

    lp_ref[...] = lp
    w_ref[...] = w
    ti_ref[...] = jnp.where(iota == 0, i1, jnp.where(iota == 1, i2, 0))
    g_ref[...] = jnp.where(iota == 0, m1, jnp.where(iota == 1, m2, 0.0))
    xb_ref[...] = x.astype(jnp.bfloat16)


def _dispatch_body(ti_ref, pos_ref, meta_ref):
    C = 512
    n = ti_ref.shape[0]
    e0 = ti_ref[:, 0:1]
    e1 = ti_ref[:, 1:2]
    lane8 = lax.broadcasted_iota(jnp.int32, (n, E), 1)
    oh0 = (e0 == lane8).astype(jnp.float32)
    oh1 = (e1 == lane8).astype(jnp.float32)
    oh = oh0 + oh1

    r = lax.broadcasted_iota(jnp.int32, (C, C), 0)
    c = lax.broadcasted_iota(jnp.int32, (C, C), 1)
    L = (r > c).astype(jnp.float32)

    carry = jnp.zeros((1, E), jnp.float32)
    chunks = []
    for k in range(n // C):
        ohk = oh[k * C:(k + 1) * C, :]
        pref = jnp.dot(L, ohk, preferred_element_type=jnp.float32) + carry
        chunks.append(pref)
        carry = carry + jnp.sum(ohk, axis=0, keepdims=True)
    prefix = jnp.concatenate(chunks, axis=0)
    counts = carry

    padded = jnp.ceil(counts / T_FFN) * T_FFN
    u_r = lax.broadcasted_iota(jnp.int32, (E, E), 0)
    u_c = lax.broadcasted_iota(jnp.int32, (E, E), 1)
    U = (u_r < u_c).astype(jnp.float32)
    starts = jnp.dot(padded, U, preferred_element_type=jnp.float32)
    ends = starts + padded

    rank0 = jnp.sum(prefix * oh0, axis=1, keepdims=True)
    rank1 = jnp.sum((prefix + oh0) * oh1, axis=1, keepdims=True)
    base0 = jnp.sum(starts * oh0, axis=1, keepdims=True)
    base1 = jnp.sum(starts * oh1, axis=1, keepdims=True)
    pos0 = (base0 + rank0).astype(jnp.int32)
    pos1 = (base1 + rank1).astype(jnp.int32)
    lane_out = lax.broadcasted_iota(jnp.int32, (n, E), 1)
    pos_ref[...] = jnp.where(lane_out == 0, pos0,
                             jnp.where(lane_out == 1, pos1, 0))

    n_tiles = (pos_ref.shape[0] * TOP_K + E * T_FFN) // T_FFN
    gl = lax.broadcasted_iota(jnp.int32, (1, 128), 1)
    ends_b = jnp.broadcast_to(ends.reshape(E, 1), (E, 128))
    te_row = jnp.minimum(
        jnp.sum((ends_b <= (gl * T_FFN).astype(jnp.float32)).astype(jnp.int32),
                axis=0, keepdims=True), E - 1)
    n_live = (ends[0, E - 1] / T_FFN).astype(jnp.int32)
    meta_ref[...] = jnp.where(gl == n_tiles, n_live, te_row)


def _ffn_body(te_ref, x_ref, w1_ref, b1_ref, w2_ref, b2_ref, y_ref):
    n_tiles = pl.num_programs(0)

    @pl.when(pl.program_id(0) < te_ref[n_tiles])
    def _():
        h = jnp.dot(x_ref[...], w1_ref[0], preferred_element_type=jnp.float32)
        h = h + b1_ref[0]
        h = 0.5 * h * (1.0 + jax.lax.erf(h * 0.7071067811865476))
        y = jnp.dot(h.astype(jnp.bfloat16), w2_ref[0],
                    preferred_element_type=jnp.float32)
        y_ref[...] = y + b2_ref[0]


def kernel(x, expert_mus, expert_log_sigmas, W1, b1, W2, b2):
    batch_size, num_tokens, _ = x.shape
    n = batch_size * num_tokens
    x_flat = x.reshape(n, D_IN)

    inv_sigmas = 1.0 / jnp.exp(expert_log_sigmas)
    lss_row = jnp.sum(expert_log_sigmas, axis=-1).reshape(1, E)

    n_rt = n // T_ROUTE
    lp, w, ti_pad, g_pad, xb = pl.pallas_call(
        _routing_body,
        grid=(n_rt,),
        in_specs=[
            pl.BlockSpec((T_ROUTE, D_IN), lambda i: (i, 0)),
            pl.BlockSpec((E, D_IN), lambda i: (0, 0)),
            pl.BlockSpec((E, D_IN), lambda i: (0, 0)),
            pl.BlockSpec((1, E), lambda i: (0, 0)),
        ],
        out_specs=[
            pl.BlockSpec((T_ROUTE, E), lambda i: (i, 0)),
            pl.BlockSpec((T_ROUTE, E), lambda i: (i, 0)),
            pl.BlockSpec((T_ROUTE, E), lambda i: (i, 0)),
            pl.BlockSpec((T_ROUTE, E), lambda i: (i, 0)),
            pl.BlockSpec((T_ROUTE, D_IN), lambda i: (i, 0)),
        ],
        out_shape=[
            jax.ShapeDtypeStruct((n, E), jnp.float32),
            jax.ShapeDtypeStruct((n, E), jnp.float32),
            jax.ShapeDtypeStruct((n, E), jnp.int32),
            jax.ShapeDtypeStruct((n, E), jnp.float32),
            jax.ShapeDtypeStruct((n, D_IN), jnp.bfloat16),
        ],
    )(x_flat, expert_mus, inv_sigmas, lss_row)

    top_indices = ti_pad[:, :TOP_K]
    gates = g_pad[:, :TOP_K]

    n_assign = n * TOP_K
    r_max = n_assign + E * T_FFN
    n_tiles = r_max // T_FFN

    pos_pad, meta_row = pl.pallas_call(
        _dispatch_body,
        out_shape=[
            jax.ShapeDtypeStruct((n, E), jnp.int32),
            jax.ShapeDtypeStruct((1, 128), jnp.int32),
        ],
    )(ti_pad)

    pos_a = pos_pad[:, :TOP_K].reshape(-1)
    token_a = jnp.arange(n_assign, dtype=jnp.int32) // TOP_K
    row_token = jnp.zeros((r_max,), jnp.int32).at[pos_a].set(token_a)
    tile_meta = meta_row[0, :n_tiles + 1]

    x_rows = xb[row_token]

    y = pl.pallas_call(
        _ffn_body,
        grid_spec=pltpu.PrefetchScalarGridSpec(
            num_scalar_prefetch=1,
            grid=(n_tiles,),
            in_specs=[
                pl.BlockSpec((T_FFN, D_IN), lambda g, te: (g, 0)),
                pl.BlockSpec((1, D_IN, D_H), lambda g, te: (te[g], 0, 0)),
                pl.BlockSpec((1, 1, D_H), lambda g, te: (te[g], 0, 0)),
                pl.BlockSpec((1, D_H, D_OUT), lambda g, te: (te[g], 0, 0)),
                pl.BlockSpec((1, 1, D_OUT), lambda g, te: (te[g], 0, 0)),
            ],
            out_specs=pl.BlockSpec((T_FFN, D_OUT), lambda g, te: (g, 0)),
        ),
        out_shape=jax.ShapeDtypeStruct((r_max, D_OUT), jnp.float32),
    )(tile_meta, x_rows, W1.astype(jnp.bfloat16), b1.reshape(E, 1, D_H),
      W2.astype(jnp.bfloat16), b2.reshape(E, 1, D_OUT))

    p0 = pos_a[0::TOP_K]
    p1 = pos_a[1::TOP_K]
    final = gates[:, 0:1] * y[p0] + gates[:, 1:2] * y[p1]

    return (final.reshape(batch_size, num_tokens, D_OUT),
            lp.reshape(batch_size, num_tokens, E),
            w.reshape(batch_size, num_tokens, E),
            top_indices)


# speedup vs baseline: 1.2491x; 1.0332x over previous
import jax
import jax.numpy as jnp
from jax import lax
from jax.experimental import pallas as pl
from jax.experimental.pallas import tpu as pltpu

E = 8
TOP_K = 2
D_IN = 1024
D_H = 4096
D_OUT = 1024

T_FFN = 256
T_ROUTE = 512


def _routing_body(x_ref, mus_ref, sig_ref, lss_ref, lp_ref, w_ref, ti_ref, g_ref,
                  xb_ref):
    x = x_ref[...]
    rows = x.shape[0]
    d = jnp.zeros((rows, E), jnp.float32)
    eidx = jax.lax.broadcasted_iota(jnp.int32, (rows, E), 1)
    for e in range(E):
        t = (x - mus_ref[pl.ds(e, 1), :]) * sig_ref[pl.ds(e, 1), :]
        de = jnp.sum(t * t, axis=1, keepdims=True)
        d = jnp.where(eidx == e, de, d)
    lp = -0.5 * d - lss_ref[...]
    m = jnp.max(lp, axis=1, keepdims=True)
    ew = jnp.exp(lp - m)
    w = ew / jnp.sum(ew, axis=1, keepdims=True)

    iota = jax.lax.broadcasted_iota(jnp.int32, w.shape, 1)
    m1 = jnp.max(w, axis=1, keepdims=True)
    i1 = jnp.min(jnp.where(w == m1, iota, E), axis=1, keepdims=True)
    wm = jnp.where(iota == i1, -jnp.inf, w)
    m2 = jnp.max(wm, axis=1, keepdims=True)
    i2 = jnp.min(jnp.where(wm == m2, iota, E), axis=1, keepdims=True)

    lp_ref[...] = lp
    w_ref[...] = w
    ti_ref[...] = jnp.where(iota == 0, i1, jnp.where(iota == 1, i2, 0))
    g_ref[...] = jnp.where(iota == 0, m1, jnp.where(iota == 1, m2, 0.0))
    xb_ref[...] = x.astype(jnp.bfloat16)


def _dispatch_body(ti_ref, pos_ref, meta_ref):
    C = 512
    n = ti_ref.shape[0]
    e0 = ti_ref[:, 0:1]
    e1 = ti_ref[:, 1:2]
    lane8 = lax.broadcasted_iota(jnp.int32, (n, E), 1)
    oh0 = (e0 == lane8).astype(jnp.float32)
    oh1 = (e1 == lane8).astype(jnp.float32)
    oh = oh0 + oh1

    r = lax.broadcasted_iota(jnp.int32, (C, C), 0)
    c = lax.broadcasted_iota(jnp.int32, (C, C), 1)
    L = (r > c).astype(jnp.float32)

    carry = jnp.zeros((1, E), jnp.float32)
    chunks = []
    for k in range(n // C):
        ohk = oh[k * C:(k + 1) * C, :]
        pref = jnp.dot(L, ohk, preferred_element_type=jnp.float32) + carry
        chunks.append(pref)
        carry = carry + jnp.sum(ohk, axis=0, keepdims=True)
    prefix = jnp.concatenate(chunks, axis=0)
    counts = carry

    padded = jnp.ceil(counts / T_FFN) * T_FFN
    u_r = lax.broadcasted_iota(jnp.int32, (E, E), 0)
    u_c = lax.broadcasted_iota(jnp.int32, (E, E), 1)
    U = (u_r < u_c).astype(jnp.float32)
    starts = jnp.dot(padded, U, preferred_element_type=jnp.float32)
    ends = starts + padded

    rank0 = jnp.sum(prefix * oh0, axis=1, keepdims=True)
    rank1 = jnp.sum((prefix + oh0) * oh1, axis=1, keepdims=True)
    base0 = jnp.sum(starts * oh0, axis=1, keepdims=True)
    base1 = jnp.sum(starts * oh1, axis=1, keepdims=True)
    pos0 = (base0 + rank0).astype(jnp.int32)
    pos1 = (base1 + rank1).astype(jnp.int32)
    lane_out = lax.broadcasted_iota(jnp.int32, (n, E), 1)
    pos_ref[...] = jnp.where(lane_out == 0, pos0,
                             jnp.where(lane_out == 1, pos1, 0))

    n_tiles = (pos_ref.shape[0] * TOP_K + E * T_FFN) // T_FFN
    gl = lax.broadcasted_iota(jnp.int32, (1, 128), 1)
    ends_b = jnp.broadcast_to(ends.reshape(E, 1), (E, 128))
    te_row = jnp.minimum(
        jnp.sum((ends_b <= (gl * T_FFN).astype(jnp.float32)).astype(jnp.int32),
                axis=0, keepdims=True), E - 1)
    n_live = (ends[0, E - 1] / T_FFN).astype(jnp.int32)
    meta_ref[...] = jnp.where(gl == n_tiles, n_live, te_row)


def _ffn_body(te_ref, x_ref, w1_ref, b1_ref, w2_ref, b2_ref, y_ref):
    n_tiles = pl.num_programs(0)

    @pl.when(pl.program_id(0) < te_ref[n_tiles])
    def _():
        h = jnp.dot(x_ref[...], w1_ref[0], preferred_element_type=jnp.float32)
        h = h + b1_ref[0]
        h = 0.5 * h * (1.0 + jax.lax.erf(h * 0.7071067811865476))
        y = jnp.dot(h.astype(jnp.bfloat16), w2_ref[0],
                    preferred_element_type=jnp.float32)
        y_ref[...] = y + b2_ref[0]


def kernel(x, expert_mus, expert_log_sigmas, W1, b1, W2, b2):
    batch_size, num_tokens, _ = x.shape
    n = batch_size * num_tokens
    x_flat = x.reshape(n, D_IN)

    inv_sigmas = 1.0 / jnp.exp(expert_log_sigmas)
    lss_row = jnp.sum(expert_log_sigmas, axis=-1).reshape(1, E)

    n_rt = n // T_ROUTE
    lp, w, ti_pad, g_pad, xb = pl.pallas_call(
        _routing_body,
        grid=(n_rt,),
        in_specs=[
            pl.BlockSpec((T_ROUTE, D_IN), lambda i: (i, 0)),
            pl.BlockSpec((E, D_IN), lambda i: (0, 0)),
            pl.BlockSpec((E, D_IN), lambda i: (0, 0)),
            pl.BlockSpec((1, E), lambda i: (0, 0)),
        ],
        out_specs=[
            pl.BlockSpec((T_ROUTE, E), lambda i: (i, 0)),
            pl.BlockSpec((T_ROUTE, E), lambda i: (i, 0)),
            pl.BlockSpec((T_ROUTE, E), lambda i: (i, 0)),
            pl.BlockSpec((T_ROUTE, E), lambda i: (i, 0)),
            pl.BlockSpec((T_ROUTE, D_IN), lambda i: (i, 0)),
        ],
        out_shape=[
            jax.ShapeDtypeStruct((n, E), jnp.float32),
            jax.ShapeDtypeStruct((n, E), jnp.float32),
            jax.ShapeDtypeStruct((n, E), jnp.int32),
            jax.ShapeDtypeStruct((n, E), jnp.float32),
            jax.ShapeDtypeStruct((n, D_IN), jnp.bfloat16),
        ],
    )(x_flat, expert_mus, inv_sigmas, lss_row)

    top_indices = ti_pad[:, :TOP_K]
    gates = g_pad[:, :TOP_K]

    n_assign = n * TOP_K
    r_max = n_assign + E * T_FFN
    n_tiles = r_max // T_FFN

    pos_pad, meta_row = pl.pallas_call(
        _dispatch_body,
        out_shape=[
            jax.ShapeDtypeStruct((n, E), jnp.int32),
            jax.ShapeDtypeStruct((1, 128), jnp.int32),
        ],
    )(ti_pad)

    pos_a = pos_pad[:, :TOP_K].reshape(-1)
    token_a = jnp.arange(n_assign, dtype=jnp.int32) // TOP_K
    row_token = jnp.zeros((r_max,), jnp.int32).at[pos_a].set(token_a)
    tile_meta = meta_row[0, :n_tiles + 1]

    x_rows = xb[row_token]

    y = pl.pallas_call(
        _ffn_body,
        grid_spec=pltpu.PrefetchScalarGridSpec(
            num_scalar_prefetch=1,
            grid=(n_tiles,),
            in_specs=[
                pl.BlockSpec((T_FFN, D_IN), lambda g, te: (g, 0)),
                pl.BlockSpec((1, D_IN, D_H), lambda g, te: (te[g], 0, 0)),
                pl.BlockSpec((1, 1, D_H), lambda g, te: (te[g], 0, 0)),
                pl.BlockSpec((1, D_H, D_OUT), lambda g, te: (te[g], 0, 0)),
                pl.BlockSpec((1, 1, D_OUT), lambda g, te: (te[g], 0, 0)),
            ],
            out_specs=pl.BlockSpec((T_FFN, D_OUT), lambda g, te: (g, 0)),
        ),
        out_shape=jax.ShapeDtypeStruct((r_max, D_OUT), jnp.float32),
    )(tile_meta, x_rows, W1.astype(jnp.bfloat16), b1.reshape(E, 1, D_H),
      W2.astype(jnp.bfloat16), b2.reshape(E, 1, D_OUT))

    p0 = pos_a[0::TOP_K]
    p1 = pos_a[1::TOP_K]
    final = gates[:, 0:1] * y[p0] + gates[:, 1:2] * y[p1]

    return (final.reshape(batch_size, num_tokens, D_OUT),
            lp.reshape(batch_size, num_tokens, E),
            w.reshape(batch_size, num_tokens, E),
            top_indices)
